# Initial kernel scaffold; baseline (speedup 1.0000x reference)
import jax
import jax.numpy as jnp
from jax.experimental import pallas as pl


def kernel(gene_x, cpg_x, mirna_x, gene_batch, cpg_batch, mirna_batch, gene_sW1, gene_sb1, gene_sW2, gene_sb2, cpg_sW1, cpg_sb1, cpg_sW2, cpg_sb2, mirna_sW1, mirna_sb1, mirna_sW2, mirna_sb2, mrna_W1, mrna_b1, mrna_W2, mrna_b2, cnv_W1, cnv_b1, cnv_W2, cnv_b2, lin_cpg_W, lin_cpg_b, lin_mir_W, lin_mir_b):
    raise NotImplementedError("write your pallas kernel here")


# speedup vs baseline: 4.3677x; 4.3677x over previous
import jax
import jax.numpy as jnp
from jax.experimental import pallas as pl
from jax.experimental.pallas import tpu as pltpu

NSEG = 64
_NEG = -1e30


def _diag(v_row):
    r = jax.lax.broadcasted_iota(jnp.int32, (NSEG, NSEG), 0)
    c = jax.lax.broadcasted_iota(jnp.int32, (NSEG, NSEG), 1)
    return jnp.where(r == c, v_row, 0.0)


def _pool_step(x_ref, b_ref, sW1_ref, sb1_ref, sW2_ref, sb2_ref,
               m_ref, d_ref, n_ref):
    i = pl.program_id(0)

    @pl.when(i == 0)
    def _init():
        m_ref[...] = jnp.full((1, NSEG), _NEG, jnp.float32)
        d_ref[...] = jnp.zeros((1, NSEG), jnp.float32)
        n_ref[...] = jnp.zeros_like(n_ref)

    x = x_ref[...]
    b = b_ref[0]
    T = x.shape[0]

    h = jnp.maximum(
        jnp.dot(x, sW1_ref[...], preferred_element_type=jnp.float32)
        + sb1_ref[...], 0.0)
    s = jnp.dot(h, sW2_ref[...], preferred_element_type=jnp.float32) \
        + sb2_ref[...]

    seg = jax.lax.broadcasted_iota(jnp.int32, (T, NSEG), 1)
    mask = (b == seg)
    masked_s = jnp.where(mask, s, _NEG)
    tmax = jnp.max(masked_s, axis=0, keepdims=True)

    m_old = m_ref[...]
    m_new = jnp.maximum(m_old, tmax)
    scale = jnp.exp(m_old - m_new)
    m_rows = jnp.sum(jnp.where(mask, m_new, 0.0), axis=1,
                     keepdims=True)
    e = jnp.exp(s - m_rows)
    we = jnp.where(mask, e, 0.0)

    m_ref[...] = m_new
    d_ref[...] = d_ref[...] * scale + jnp.sum(we, axis=0, keepdims=True)
    npart = jax.lax.dot_general(we, x, (((0,), (0,)), ((), ())),
                                preferred_element_type=jnp.float32)
    n_ref[...] = jnp.dot(_diag(scale), n_ref[...],
                         preferred_element_type=jnp.float32) + npart


def _pooled(d_ref, n_ref):
    d = d_ref[...]
    r = jnp.where(d > 0, 1.0 / jnp.maximum(d, 1e-30), 0.0)
    return jnp.dot(_diag(r), n_ref[...],
                   preferred_element_type=jnp.float32)


def _gene_body(x_ref, b_ref, sW1_ref, sb1_ref, sW2_ref, sb2_ref,
               mW1_ref, mb1_ref, mW2_ref, mb2_ref,
               cW1_ref, cb1_ref, cW2_ref, cb2_ref,
               z1_ref, z2_ref, m_ref, d_ref, n_ref):
    _pool_step(x_ref, b_ref, sW1_ref, sb1_ref, sW2_ref, sb2_ref,
               m_ref, d_ref, n_ref)

    @pl.when(pl.program_id(0) == pl.num_programs(0) - 1)
    def _heads():
        g = _pooled(d_ref, n_ref)
        h1 = jnp.maximum(
            jnp.dot(g, mW1_ref[...], preferred_element_type=jnp.float32)
            + mb1_ref[...], 0.0)
        z1_ref[...] = jnp.dot(h1, mW2_ref[...],
                              preferred_element_type=jnp.float32) + mb2_ref[...]
        h2 = jnp.maximum(
            jnp.dot(g, cW1_ref[...], preferred_element_type=jnp.float32)
            + cb1_ref[...], 0.0)
        z2_ref[...] = jnp.dot(h2, cW2_ref[...],
                              preferred_element_type=jnp.float32) + cb2_ref[...]


def _lin_body(x_ref, b_ref, sW1_ref, sb1_ref, sW2_ref, sb2_ref,
              W_ref, bb_ref, z_ref, m_ref, d_ref, n_ref):
    _pool_step(x_ref, b_ref, sW1_ref, sb1_ref, sW2_ref, sb2_ref,
               m_ref, d_ref, n_ref)

    @pl.when(pl.program_id(0) == pl.num_programs(0) - 1)
    def _heads():
        g = _pooled(d_ref, n_ref)
        z_ref[...] = jnp.dot(g, W_ref[...],
                             preferred_element_type=jnp.float32) + bb_ref[...]


def _full(shape):
    return pl.BlockSpec(shape, lambda i: tuple(0 for _ in shape))


def _attn_pool_call(x, batch, sW1, sb1, sW2, sb2, head_args, body, n_out, T):
    N, D = x.shape
    G = N // T
    batch3 = batch.reshape(G, T, 1)
    in_specs = [
        pl.BlockSpec((T, D), lambda i: (i, 0)),
        pl.BlockSpec((1, T, 1), lambda i: (i, 0, 0)),
        _full((D, NSEG)), _full((1, NSEG)), _full((NSEG, 1)), _full((1, 1)),
    ] + [_full(a.shape) for a in head_args]
    out_specs = [pl.BlockSpec((NSEG, 128), lambda i: (0, 0))] * n_out
    out_shapes = [jax.ShapeDtypeStruct((NSEG, 128), jnp.float32)] * n_out
    outs = pl.pallas_call(
        body,
        grid=(G,),
        in_specs=in_specs,
        out_specs=out_specs if n_out > 1 else out_specs[0],
        out_shape=out_shapes if n_out > 1 else out_shapes[0],
        scratch_shapes=[
            pltpu.VMEM((1, NSEG), jnp.float32),
            pltpu.VMEM((1, NSEG), jnp.float32),
            pltpu.VMEM((NSEG, D), jnp.float32),
        ],
    )(x, batch3, sW1, sb1.reshape(1, NSEG), sW2, sb2.reshape(1, 1),
      *head_args)
    return outs


def kernel(gene_x, cpg_x, mirna_x, gene_batch, cpg_batch, mirna_batch,
           gene_sW1, gene_sb1, gene_sW2, gene_sb2,
           cpg_sW1, cpg_sb1, cpg_sW2, cpg_sb2,
           mirna_sW1, mirna_sb1, mirna_sW2, mirna_sb2,
           mrna_W1, mrna_b1, mrna_W2, mrna_b2,
           cnv_W1, cnv_b1, cnv_W2, cnv_b2,
           lin_cpg_W, lin_cpg_b, lin_mir_W, lin_mir_b):
    z_mrna, z_cnv = _attn_pool_call(
        gene_x, gene_batch, gene_sW1, gene_sb1, gene_sW2, gene_sb2,
        (mrna_W1, mrna_b1.reshape(1, -1), mrna_W2, mrna_b2.reshape(1, -1),
         cnv_W1, cnv_b1.reshape(1, -1), cnv_W2, cnv_b2.reshape(1, -1)),
        _gene_body, 2, T=2000)
    z_dnam = _attn_pool_call(
        cpg_x, cpg_batch, cpg_sW1, cpg_sb1, cpg_sW2, cpg_sb2,
        (lin_cpg_W, lin_cpg_b.reshape(1, -1)),
        _lin_body, 1, T=2000)
    z_mir = _attn_pool_call(
        mirna_x, mirna_batch, mirna_sW1, mirna_sb1, mirna_sW2, mirna_sb2,
        (lin_mir_W, lin_mir_b.reshape(1, -1)),
        _lin_body, 1, T=2000)
    return (z_mrna, z_cnv, z_dnam, z_mir)
